# baseline (speedup 1.0000x reference)
import functools
import jax
import jax.numpy as jnp
from jax import lax
from jax.experimental import pallas as pl
from jax.experimental.pallas import tpu as pltpu
from jax.experimental.pallas import tpu_sc as plsc

B, C, HW, P, OUT = 16384, 11, 64, 32, 64
NW = 32
BPT = B // NW
BC = 128
NCHUNK = BPT // BC
NLG = BC // 16
PB = 8


def _sc_kernel_body(ids_hbm, board_hbm, raw_hbm, ids_v, board_v, table_v,
                    raw_a, raw_b, sem_ids, sem_board, sem_out):
    wid = lax.axis_index("s") * 2 + lax.axis_index("c")
    lanes = lax.iota(jnp.int32, 16)
    sent = jnp.full((16,), HW, jnp.int32)

    def chunk(g, _):
        cb = pl.multiple_of(wid * BPT + g * BC, BC)
        cp_ids = pltpu.async_copy(ids_hbm.at[:, pl.ds(cb, BC)], ids_v,
                                  sem_ids)
        cp_board = pltpu.async_copy(board_hbm.at[:, pl.ds(cb, BC)], board_v,
                                    sem_board)

        for t in range(33):
            for lg in range(NLG):
                table_v[t, pl.ds(lg * 16, 16)] = sent
        cp_ids.wait()

        def scan_cell(j, _):
            jj = 63 - j
            jjv = jnp.full((16,), jj, jnp.int32)
            for lg in range(NLG):
                col = lanes + lg * 16
                ids_vec = plsc.load_gather(ids_v, [jjv, col])
                plsc.store_scatter(table_v, [ids_vec, col], jjv)
            return _
        lax.fori_loop(0, HW, scan_cell, None)
        cp_board.wait()

        out_cps = {}
        for pb in range(P // PB):
            buf = raw_a if pb % 2 == 0 else raw_b
            if pb >= 2:
                out_cps[pb - 2].wait()

            @plsc.parallel_loop(0, PB, 1, unroll=4)
            def piece(p):
                tv = jnp.full((16,), pb * PB + 1, jnp.int32) + p
                pv = jnp.zeros((16,), jnp.int32) + p
                for lg in range(NLG):
                    col = lanes + lg * 16
                    fvec = plsc.load_gather(table_v, [tv, col])
                    msk = fvec < HW
                    f0 = jnp.where(msk, fvec, 0)
                    for c in range(C):
                        val = plsc.load_gather(board_v, [f0 + c * HW, col])
                        val = jnp.where(msk, val, 0.0)
                        plsc.store_scatter(buf, [pv * C + c, col], val)

            out_cps[pb] = pltpu.async_copy(
                buf, raw_hbm.at[pl.ds(pb * PB * C, PB * C), pl.ds(cb, BC)],
                sem_out)
        out_cps[2].wait()
        out_cps[3].wait()
        return _
    lax.fori_loop(0, NCHUNK, chunk, None)


def _sc_extract(ids_t, board_t):
    mesh = plsc.VectorSubcoreMesh(core_axis_name="c", subcore_axis_name="s")
    k = functools.partial(
        pl.kernel,
        mesh=mesh,
        compiler_params=pltpu.CompilerParams(needs_layout_passes=False),
        out_type=jax.ShapeDtypeStruct((P * C, B), jnp.float32),
        scratch_types=[
            pltpu.VMEM((HW, BC), jnp.int32),
            pltpu.VMEM((C * HW, BC), jnp.float32),
            pltpu.VMEM((33, BC), jnp.int32),
            pltpu.VMEM((PB * C, BC), jnp.float32),
            pltpu.VMEM((PB * C, BC), jnp.float32),
            pltpu.SemaphoreType.DMA,
            pltpu.SemaphoreType.DMA,
            pltpu.SemaphoreType.DMA,
        ],
    )(_sc_kernel_body)
    return k(ids_t, board_t)


def _tc_body(raw_ref, w_ref, bias_ref, out_ref):
    out_ref[0] = jnp.dot(w_ref[...], raw_ref[0],
                         preferred_element_type=jnp.float32) + bias_ref[...]


BN = 16384


@jax.jit
def kernel(full_board_vector, piece_ids, proj_w, proj_b):
    ids_t = piece_ids.transpose(1, 2, 0).reshape(HW, B)
    board_t = full_board_vector.transpose(1, 2, 3, 0).reshape(C * HW, B)

    raw = _sc_extract(ids_t, board_t).reshape(P, C, B)
    bias = proj_b.reshape(OUT, 1)

    out = pl.pallas_call(
        _tc_body,
        grid=(P, B // BN),
        in_specs=[
            pl.BlockSpec((1, C, BN), lambda p, i: (p, 0, i)),
            pl.BlockSpec((OUT, C), lambda p, i: (0, 0)),
            pl.BlockSpec((OUT, 1), lambda p, i: (0, 0)),
        ],
        out_specs=pl.BlockSpec((1, OUT, BN), lambda p, i: (p, 0, i)),
        out_shape=jax.ShapeDtypeStruct((P, OUT, B), jnp.float32),
    )(raw, proj_w, bias)
    return out.transpose(2, 0, 1)


# speedup vs baseline: 1.2644x; 1.2644x over previous
import functools
import jax
import jax.numpy as jnp
from jax import lax
from jax.experimental import pallas as pl
from jax.experimental.pallas import tpu as pltpu
from jax.experimental.pallas import tpu_sc as plsc

B, C, HW, P, OUT = 16384, 11, 64, 32, 64
NW = 32
BPT = B // NW
BC = 128
NCHUNK = BPT // BC
NLG = BC // 16
PB = 8


NSPLIT = 2
BH = B // NSPLIT
BPTH = BPT // NSPLIT


def _sc_kernel_body(h, ids_hbm, board_hbm, raw_hbm, ids_v, board_v, table_v,
                    raw_a, raw_b, sem_ids, sem_board, sem_out):
    wid = lax.axis_index("s") * 2 + lax.axis_index("c")
    lanes = lax.iota(jnp.int32, 16)
    sent = jnp.full((16,), HW, jnp.int32)

    def chunk(g, _):
        cb = pl.multiple_of(wid * BPTH + g * BC, BC)
        cin = pl.multiple_of(cb + h * BH, BC)
        cp_ids = pltpu.async_copy(ids_hbm.at[:, pl.ds(cin, BC)], ids_v,
                                  sem_ids)
        cp_board = pltpu.async_copy(board_hbm.at[:, pl.ds(cin, BC)], board_v,
                                    sem_board)

        for t in range(33):
            for lg in range(NLG):
                table_v[t, pl.ds(lg * 16, 16)] = sent
        cp_ids.wait()

        def scan_cell(j, _):
            jj = 63 - j
            jjv = jnp.full((16,), jj, jnp.int32)
            for lg in range(NLG):
                col = lanes + lg * 16
                ids_vec = plsc.load_gather(ids_v, [jjv, col])
                plsc.store_scatter(table_v, [ids_vec, col], jjv)
            return _
        lax.fori_loop(0, HW, scan_cell, None)
        cp_board.wait()

        out_cps = {}
        for pb in range(P // PB):
            buf = raw_a if pb % 2 == 0 else raw_b
            if pb >= 2:
                out_cps[pb - 2].wait()

            @plsc.parallel_loop(0, PB, 1, unroll=2)
            def piece(p):
                tv = jnp.full((16,), pb * PB + 1, jnp.int32) + p
                pv = jnp.zeros((16,), jnp.int32) + p
                for lg in range(NLG):
                    col = lanes + lg * 16
                    fvec = plsc.load_gather(table_v, [tv, col])
                    msk = fvec < HW
                    f0 = jnp.where(msk, fvec, 0)
                    for c in range(C):
                        val = plsc.load_gather(board_v, [f0 + c * HW, col])
                        val = jnp.where(msk, val, 0.0)
                        plsc.store_scatter(buf, [pv * C + c, col], val)

            out_cps[pb] = pltpu.async_copy(
                buf, raw_hbm.at[pl.ds(pb * PB * C, PB * C), pl.ds(cb, BC)],
                sem_out)
        out_cps[2].wait()
        out_cps[3].wait()
        return _
    lax.fori_loop(0, BPTH // BC, chunk, None)


def _sc_extract(ids_t, board_t, h):
    mesh = plsc.VectorSubcoreMesh(core_axis_name="c", subcore_axis_name="s")
    k = functools.partial(
        pl.kernel,
        mesh=mesh,
        compiler_params=pltpu.CompilerParams(needs_layout_passes=False),
        out_type=jax.ShapeDtypeStruct((P * C, BH), jnp.float32),
        scratch_types=[
            pltpu.VMEM((HW, BC), jnp.int32),
            pltpu.VMEM((C * HW, BC), jnp.float32),
            pltpu.VMEM((33, BC), jnp.int32),
            pltpu.VMEM((PB * C, BC), jnp.float32),
            pltpu.VMEM((PB * C, BC), jnp.float32),
            pltpu.SemaphoreType.DMA,
            pltpu.SemaphoreType.DMA,
            pltpu.SemaphoreType.DMA,
        ],
        name=f"sc_extract_h{h}",
    )(functools.partial(_sc_kernel_body, h))
    return k(ids_t, board_t)


def _tc_body0(raw_ref, w_ref, bias_ref, out_ref):
    out_ref[0] = jnp.dot(w_ref[...], raw_ref[0],
                         preferred_element_type=jnp.float32) + bias_ref[...]


def _tc_body1(raw_ref, w_ref, bias_ref, prev_ref, out_ref):
    out_ref[0] = jnp.dot(w_ref[...], raw_ref[0],
                         preferred_element_type=jnp.float32) + bias_ref[...]


def _tc_project(raw, proj_w, bias, h, prev=None):
    common = dict(
        grid=(P,),
        out_specs=pl.BlockSpec((1, OUT, BH), lambda p: (p, 0, h)),
        out_shape=jax.ShapeDtypeStruct((P, OUT, B), jnp.float32),
    )
    in_specs = [
        pl.BlockSpec((1, C, BH), lambda p: (p, 0, 0)),
        pl.BlockSpec((OUT, C), lambda p: (0, 0)),
        pl.BlockSpec((OUT, 1), lambda p: (0, 0)),
    ]
    if prev is None:
        return pl.pallas_call(_tc_body0, in_specs=in_specs, **common)(
            raw, proj_w, bias)
    in_specs.append(pl.BlockSpec(memory_space=pl.ANY))
    return pl.pallas_call(_tc_body1, in_specs=in_specs,
                          input_output_aliases={3: 0}, **common)(
        raw, proj_w, bias, prev)


@jax.jit
def kernel(full_board_vector, piece_ids, proj_w, proj_b):
    ids_t = piece_ids.transpose(1, 2, 0).reshape(HW, B)
    board_t = full_board_vector.transpose(1, 2, 3, 0).reshape(C * HW, B)

    bias = proj_b.reshape(OUT, 1)
    raw0 = _sc_extract(ids_t, board_t, 0).reshape(P, C, BH)
    raw1 = _sc_extract(ids_t, board_t, 1).reshape(P, C, BH)
    out = _tc_project(raw0, proj_w, bias, 0)
    out = _tc_project(raw1, proj_w, bias, 1, prev=out)
    return out.transpose(2, 0, 1)


# speedup vs baseline: 1.4981x; 1.1848x over previous
import functools
import jax
import jax.numpy as jnp
from jax import lax
from jax.experimental import pallas as pl
from jax.experimental.pallas import tpu as pltpu
from jax.experimental.pallas import tpu_sc as plsc

B, C, HW, P, OUT = 16384, 11, 64, 32, 64
NW = 32
BPT = B // NW
BC = 128
NCHUNK = BPT // BC
NLG = BC // 16
PB = 8


NSPLIT = 2
BH = B // NSPLIT
BPTH = BPT // NSPLIT


def _sc_kernel_body(h, ids_hbm, board_hbm, raw_hbm, ids_v, board_v, table_v,
                    raw_a, raw_b, sem_ids, sem_board, sem_out):
    wid = lax.axis_index("s") * 2 + lax.axis_index("c")
    lanes = lax.iota(jnp.int32, 16)
    sent = jnp.full((16,), HW, jnp.int32)

    def chunk(g, _):
        cb = pl.multiple_of(wid * BPTH + g * BC, BC)
        cin = pl.multiple_of(cb + h * BH, BC)
        cp_ids = pltpu.async_copy(ids_hbm.at[:, pl.ds(cin, BC)], ids_v,
                                  sem_ids)
        cp_board = pltpu.async_copy(board_hbm.at[:, pl.ds(cin, BC)], board_v,
                                    sem_board)

        for t in range(33):
            for lg in range(NLG):
                table_v[t, pl.ds(lg * 16, 16)] = sent
        cp_ids.wait()

        def scan_cell(j, _):
            jj = 63 - j
            jjv = jnp.full((16,), jj, jnp.int32)
            for lg in range(NLG):
                col = lanes + lg * 16
                ids_vec = plsc.load_gather(ids_v, [jjv, col])
                plsc.store_scatter(table_v, [ids_vec, col], jjv)
            return _
        lax.fori_loop(0, HW, scan_cell, None)
        cp_board.wait()

        out_cps = {}
        for pb in range(P // PB):
            buf = raw_a if pb % 2 == 0 else raw_b
            if pb >= 2:
                out_cps[pb - 2].wait()

            @plsc.parallel_loop(0, PB, 1, unroll=2)
            def piece(p):
                tv = jnp.full((16,), pb * PB + 1, jnp.int32) + p
                pv = jnp.zeros((16,), jnp.int32) + p
                for lg in range(NLG):
                    col = lanes + lg * 16
                    fvec = plsc.load_gather(table_v, [tv, col])
                    msk = fvec < HW
                    f0 = jnp.where(msk, fvec, 0)
                    for c in range(C):
                        val = plsc.load_gather(board_v, [f0 + c * HW, col])
                        val = jnp.where(msk, val, 0.0)
                        plsc.store_scatter(buf, [pv * C + c, col], val)

            out_cps[pb] = pltpu.async_copy(
                buf, raw_hbm.at[pl.ds(pb * PB * C, PB * C), pl.ds(cb, BC)],
                sem_out)
        out_cps[2].wait()
        out_cps[3].wait()
        return _
    lax.fori_loop(0, BPTH // BC, chunk, None)


def _sc_extract(ids_t, board_t, h):
    mesh = plsc.VectorSubcoreMesh(core_axis_name="c", subcore_axis_name="s")
    k = functools.partial(
        pl.kernel,
        mesh=mesh,
        compiler_params=pltpu.CompilerParams(needs_layout_passes=False),
        out_type=jax.ShapeDtypeStruct((P * C, BH), jnp.float32),
        scratch_types=[
            pltpu.VMEM((HW, BC), jnp.int32),
            pltpu.VMEM((C * HW, BC), jnp.float32),
            pltpu.VMEM((33, BC), jnp.int32),
            pltpu.VMEM((PB * C, BC), jnp.float32),
            pltpu.VMEM((PB * C, BC), jnp.float32),
            pltpu.SemaphoreType.DMA,
            pltpu.SemaphoreType.DMA,
            pltpu.SemaphoreType.DMA,
        ],
        name=f"sc_extract_h{h}",
    )(functools.partial(_sc_kernel_body, h))
    return k(ids_t, board_t)


def _tc_body0(raw_ref, w_ref, bias_ref, out_ref):
    out_ref[...] = jnp.dot(w_ref[...], raw_ref[...],
                           preferred_element_type=jnp.float32) + bias_ref[...]


def _tc_body1(raw_ref, w_ref, bias_ref, prev_ref, out_ref):
    out_ref[...] = jnp.dot(w_ref[...], raw_ref[...],
                           preferred_element_type=jnp.float32) + bias_ref[...]


PG = 8
BN = 4096


def _tc_project(raw, wbig, bias_big, h, prev=None):
    nb = BH // BN
    common = dict(
        grid=(P // PG, nb),
        out_specs=pl.BlockSpec((PG * OUT, BN), lambda g, i: (g, i + h * nb)),
        out_shape=jax.ShapeDtypeStruct((P * OUT, B), jnp.float32),
    )
    in_specs = [
        pl.BlockSpec((PG * C, BN), lambda g, i: (g, i)),
        pl.BlockSpec((PG * OUT, PG * C), lambda g, i: (0, 0)),
        pl.BlockSpec((PG * OUT, 1), lambda g, i: (0, 0)),
    ]
    if prev is None:
        return pl.pallas_call(_tc_body0, in_specs=in_specs, **common)(
            raw, wbig, bias_big)
    in_specs.append(pl.BlockSpec(memory_space=pl.ANY))
    return pl.pallas_call(_tc_body1, in_specs=in_specs,
                          input_output_aliases={3: 0}, **common)(
        raw, wbig, bias_big, prev)


@jax.jit
def kernel(full_board_vector, piece_ids, proj_w, proj_b):
    ids_t = piece_ids.transpose(1, 2, 0).reshape(HW, B)
    board_t = full_board_vector.transpose(1, 2, 3, 0).reshape(C * HW, B)

    wbig = jnp.kron(jnp.eye(PG, dtype=jnp.float32), proj_w)
    bias_big = jnp.tile(proj_b, PG).reshape(PG * OUT, 1)

    raw0 = _sc_extract(ids_t, board_t, 0)
    raw1 = _sc_extract(ids_t, board_t, 1)
    out = _tc_project(raw0, wbig, bias_big, 0)
    out = _tc_project(raw1, wbig, bias_big, 1, prev=out)
    return out.reshape(P, OUT, B).transpose(2, 0, 1)


# speedup vs baseline: 2.0692x; 1.3812x over previous
import functools
import jax
import jax.numpy as jnp
from jax import lax
from jax.experimental import pallas as pl
from jax.experimental.pallas import tpu as pltpu
from jax.experimental.pallas import tpu_sc as plsc

B, C, HW, P, OUT = 16384, 11, 64, 32, 64
NW = 32
BPT = B // NW
BC = 128
NCHUNK = BPT // BC
NLG = BC // 16
PB = 8


NSPLIT = 2
BH = B // NSPLIT
BPTH = BPT // NSPLIT


def _sc_kernel_body(h, ids_hbm, board_hbm, raw_hbm, ids_v, board_v, table_v,
                    raw_a, raw_b, sem_ids, sem_board, sem_out):
    wid = lax.axis_index("s") * 2 + lax.axis_index("c")
    lanes = lax.iota(jnp.int32, 16)
    sent = jnp.full((16,), HW, jnp.int32)

    def chunk(g, _):
        cb = pl.multiple_of(wid * BPTH + g * BC, BC)
        cin = pl.multiple_of(cb + h * BH, BC)
        cp_ids = pltpu.async_copy(ids_hbm.at[:, pl.ds(cin, BC)], ids_v,
                                  sem_ids)
        cp_board = pltpu.async_copy(board_hbm.at[:, pl.ds(cin, BC)], board_v,
                                    sem_board)

        for t in range(33):
            for lg in range(NLG):
                table_v[t, pl.ds(lg * 16, 16)] = sent
        cp_ids.wait()

        def scan_cell(j, _):
            jj = 63 - j
            jjv = jnp.full((16,), jj, jnp.int32)
            for lg in range(NLG):
                col = lanes + lg * 16
                ids_vec = plsc.load_gather(ids_v, [jjv, col])
                plsc.store_scatter(table_v, [ids_vec, col], jjv)
            return _
        lax.fori_loop(0, HW, scan_cell, None)
        cp_board.wait()

        out_cps = {}
        for pb in range(P // PB):
            buf = raw_a if pb % 2 == 0 else raw_b
            if pb >= 2:
                out_cps[pb - 2].wait()

            @plsc.parallel_loop(0, PB * NLG, 1, unroll=4)
            def piece(i):
                p = i >> 3
                lg = i & (NLG - 1)
                tv = jnp.full((16,), pb * PB + 1, jnp.int32) + p
                pv = jnp.zeros((16,), jnp.int32) + p
                col = lanes + lg * 16
                fvec = plsc.load_gather(table_v, [tv, col])
                msk = fvec < HW
                f0 = jnp.where(msk, fvec, 0)
                for c in range(C):
                    val = plsc.load_gather(board_v, [f0 + c * HW, col])
                    val = jnp.where(msk, val, 0.0)
                    plsc.store_scatter(buf, [pv * C + c, col], val)

            out_cps[pb] = pltpu.async_copy(
                buf, raw_hbm.at[pl.ds(pb * PB * C, PB * C), pl.ds(cb, BC)],
                sem_out)
        out_cps[2].wait()
        out_cps[3].wait()
        return _
    lax.fori_loop(0, BPTH // BC, chunk, None)


def _sc_extract(ids_t, board_t, h):
    mesh = plsc.VectorSubcoreMesh(core_axis_name="c", subcore_axis_name="s")
    k = functools.partial(
        pl.kernel,
        mesh=mesh,
        compiler_params=pltpu.CompilerParams(needs_layout_passes=False),
        out_type=jax.ShapeDtypeStruct((P * C, BH), jnp.float32),
        scratch_types=[
            pltpu.VMEM((HW, BC), jnp.int32),
            pltpu.VMEM((C * HW, BC), jnp.float32),
            pltpu.VMEM((33, BC), jnp.int32),
            pltpu.VMEM((PB * C, BC), jnp.float32),
            pltpu.VMEM((PB * C, BC), jnp.float32),
            pltpu.SemaphoreType.DMA,
            pltpu.SemaphoreType.DMA,
            pltpu.SemaphoreType.DMA,
        ],
        name=f"sc_extract_h{h}",
    )(functools.partial(_sc_kernel_body, h))
    return k(ids_t, board_t)


def _tc_body0(raw_ref, w_ref, bias_ref, out_ref):
    out_ref[...] = jnp.dot(w_ref[...], raw_ref[...],
                           preferred_element_type=jnp.float32) + bias_ref[...]


def _tc_body1(raw_ref, w_ref, bias_ref, prev_ref, out_ref):
    out_ref[...] = jnp.dot(w_ref[...], raw_ref[...],
                           preferred_element_type=jnp.float32) + bias_ref[...]


PG = 8
BN = 4096


def _tc_project(raw, wbig, bias_big, h, prev=None):
    nb = BH // BN
    common = dict(
        grid=(P // PG, nb),
        out_specs=pl.BlockSpec((PG * OUT, BN), lambda g, i: (g, i + h * nb)),
        out_shape=jax.ShapeDtypeStruct((P * OUT, B), jnp.float32),
    )
    in_specs = [
        pl.BlockSpec((PG * C, BN), lambda g, i: (g, i)),
        pl.BlockSpec((PG * OUT, PG * C), lambda g, i: (0, 0)),
        pl.BlockSpec((PG * OUT, 1), lambda g, i: (0, 0)),
    ]
    if prev is None:
        return pl.pallas_call(_tc_body0, in_specs=in_specs, **common)(
            raw, wbig, bias_big)
    in_specs.append(pl.BlockSpec(memory_space=pl.ANY))
    return pl.pallas_call(_tc_body1, in_specs=in_specs,
                          input_output_aliases={3: 0}, **common)(
        raw, wbig, bias_big, prev)


@jax.jit
def kernel(full_board_vector, piece_ids, proj_w, proj_b):
    ids_t = piece_ids.transpose(1, 2, 0).reshape(HW, B)
    board_t = full_board_vector.transpose(1, 2, 3, 0).reshape(C * HW, B)

    wbig = jnp.kron(jnp.eye(PG, dtype=jnp.float32), proj_w)
    bias_big = jnp.tile(proj_b, PG).reshape(PG * OUT, 1)

    raw0 = _sc_extract(ids_t, board_t, 0)
    raw1 = _sc_extract(ids_t, board_t, 1)
    out = _tc_project(raw0, wbig, bias_big, 0)
    out = _tc_project(raw1, wbig, bias_big, 1, prev=out)
    return out.reshape(P, OUT, B).transpose(2, 0, 1)
